# Initial kernel scaffold; baseline (speedup 1.0000x reference)
import jax
import jax.numpy as jnp
from jax.experimental import pallas as pl


def kernel(x, ind):
    raise NotImplementedError("write your pallas kernel here")


# speedup vs baseline: 64.1595x; 64.1595x over previous
import functools

import jax
import jax.numpy as jnp
from jax import lax
from jax.experimental import pallas as pl
from jax.experimental.pallas import tpu as pltpu
from jax.experimental.pallas import tpu_sc as plsc

L = 16


def _unpool_body(x_hbm, ind_hbm, out_hbm, ind_v, x_v, out_v):
    n_pairs, hw = ind_hbm.shape
    ohw = out_hbm.shape[1]
    n_rows = x_hbm.shape[0]
    t_steps = n_rows // n_pairs
    c_dim = 96
    b_stride = t_steps * c_dim

    w = lax.axis_index("s") * 2 + lax.axis_index("c")
    pairs_per_w = n_pairs // 32

    nvreg = hw // L
    nz = ohw // L
    zeros16 = jnp.zeros((L,), jnp.float32)

    def zbody(i, _):
        out_v[pl.ds(i * L, L)] = zeros16
        return 0

    lax.fori_loop(0, nz, zbody, 0)

    def scatter_vals(i, _):
        idx = ind_v[pl.ds(i * L, L)]
        val = x_v[pl.ds(i * L, L)]
        plsc.store_scatter(out_v, [idx], val)
        return 0

    def scatter_zeros(i, _):
        idx = ind_v[pl.ds(i * L, L)]
        plsc.store_scatter(out_v, [idx], zeros16)
        return 0

    for k in range(pairs_per_w):
        p = w * pairs_per_w + k
        b = p // c_dim
        c = p - b * c_dim
        pltpu.sync_copy(ind_hbm.at[p], ind_v)
        for t in range(t_steps):
            r = b * b_stride + t * c_dim + c
            pltpu.sync_copy(x_hbm.at[r], x_v)
            lax.fori_loop(0, nvreg, scatter_vals, 0)
            pltpu.sync_copy(out_v, out_hbm.at[r])
        if k != pairs_per_w - 1:
            lax.fori_loop(0, nvreg, scatter_zeros, 0)


@jax.jit
def kernel(x, ind):
    bb, tt, cc, h, ww = x.shape
    hw = h * ww
    ohw = 4 * hw
    x2 = x.reshape(bb * tt * cc, hw)
    ind2 = ind.reshape(bb * cc, hw)
    mesh = plsc.VectorSubcoreMesh(
        core_axis_name="c", subcore_axis_name="s", num_cores=2, num_subcores=16
    )
    run = pl.kernel(
        functools.partial(_unpool_body),
        out_type=jax.ShapeDtypeStruct((bb * tt * cc, ohw), jnp.float32),
        mesh=mesh,
        scratch_types=[
            pltpu.VMEM((hw,), jnp.int32),
            pltpu.VMEM((hw,), jnp.float32),
            pltpu.VMEM((ohw,), jnp.float32),
        ],
        compiler_params=pltpu.CompilerParams(needs_layout_passes=False),
    )
    out = run(x2, ind2)
    return out.reshape(bb, tt, cc, 2 * h, 2 * ww)


# speedup vs baseline: 79.7498x; 1.2430x over previous
import jax
import jax.numpy as jnp
from jax import lax
from jax.experimental import pallas as pl
from jax.experimental.pallas import tpu as pltpu
from jax.experimental.pallas import tpu_sc as plsc

L = 16
C_DIM = 96
N_WORKERS = 32


def _unpool_body(x_hbm, ind_hbm, out_hbm, ind_v, x_v, out_a, out_b,
                 s_ind, s_x, s_oa, s_ob):
    n_pairs, hw = ind_hbm.shape
    ohw = out_a.shape[0]
    t_stride = C_DIM
    b_stride = 2 * C_DIM

    w = lax.axis_index("s") * 2 + lax.axis_index("c")
    pairs_per_w = n_pairs // N_WORKERS

    nvreg = hw // L
    nz = ohw // L
    zeros16 = jnp.zeros((L,), jnp.float32)

    def memset_plane(ref):
        def body(i, _):
            base = i * (8 * L)
            for u in range(8):
                ref[pl.ds(base + u * L, L)] = zeros16
            return 0

        lax.fori_loop(0, nz // 8, body, 0)

    def scatter_plane(ref):
        def body(i, _):
            base = i * (4 * L)
            for u in range(4):
                o = base + u * L
                idx = ind_v[pl.ds(o, L)]
                val = x_v[pl.ds(o, L)]
                plsc.store_scatter(ref, [idx], val)
            return 0

        lax.fori_loop(0, nvreg // 4, body, 0)

    def rows(k):
        p = w * pairs_per_w + k
        b = p // C_DIM
        c = p - b * C_DIM
        r0 = b * b_stride + c
        return p, r0, r0 + t_stride

    p0, r0_0, _ = rows(0)
    h_ind = pltpu.async_copy(ind_hbm.at[p0], ind_v, s_ind)
    h_x = pltpu.async_copy(x_hbm.at[r0_0], x_v, s_x)
    memset_plane(out_a)
    memset_plane(out_b)
    h_oa = None
    h_ob = None

    for k in range(pairs_per_w):
        _, r0, r1 = rows(k)
        h_ind.wait()
        h_x.wait()
        if h_oa is not None:
            h_oa.wait()
            memset_plane(out_a)
        scatter_plane(out_a)
        h_oa = pltpu.async_copy(out_a, out_hbm.at[r0], s_oa)
        h_x = pltpu.async_copy(x_hbm.at[r1], x_v, s_x)
        if h_ob is not None:
            h_ob.wait()
            memset_plane(out_b)
        h_x.wait()
        scatter_plane(out_b)
        h_ob = pltpu.async_copy(out_b, out_hbm.at[r1], s_ob)
        if k + 1 < pairs_per_w:
            p_n, r0_n, _ = rows(k + 1)
            h_x = pltpu.async_copy(x_hbm.at[r0_n], x_v, s_x)
            h_ind = pltpu.async_copy(ind_hbm.at[p_n], ind_v, s_ind)

    h_oa.wait()
    h_ob.wait()


@jax.jit
def kernel(x, ind):
    bb, tt, cc, h, ww = x.shape
    hw = h * ww
    ohw = 4 * hw
    x2 = x.reshape(bb * tt * cc, hw)
    ind2 = ind.reshape(bb * cc, hw)
    mesh = plsc.VectorSubcoreMesh(
        core_axis_name="c", subcore_axis_name="s", num_cores=2, num_subcores=16
    )
    run = pl.kernel(
        _unpool_body,
        out_type=jax.ShapeDtypeStruct((bb * tt * cc, ohw), jnp.float32),
        mesh=mesh,
        scratch_types=[
            pltpu.VMEM((hw,), jnp.int32),
            pltpu.VMEM((hw,), jnp.float32),
            pltpu.VMEM((ohw,), jnp.float32),
            pltpu.VMEM((ohw,), jnp.float32),
            pltpu.SemaphoreType.DMA,
            pltpu.SemaphoreType.DMA,
            pltpu.SemaphoreType.DMA,
            pltpu.SemaphoreType.DMA,
        ],
        compiler_params=pltpu.CompilerParams(needs_layout_passes=False),
    )
    out = run(x2, ind2)
    return out.reshape(bb, tt, cc, 2 * h, 2 * ww)


# speedup vs baseline: 95.8834x; 1.2023x over previous
import jax
import jax.numpy as jnp
from jax import lax
from jax.experimental import pallas as pl
from jax.experimental.pallas import tpu as pltpu
from jax.experimental.pallas import tpu_sc as plsc

L = 16
C_DIM = 96
N_WORKERS = 32
CHUNK = 56


def _unpool_body(x_hbm, ind_hbm, out_hbm,
                 ind_c, x_c, out_a, out_b,
                 s_i, s_x, s_oa, s_ob):
    n_pairs, h, w = ind_hbm.shape
    oh, ow = out_a.shape
    t_stride = C_DIM
    b_stride = 2 * C_DIM

    wid = lax.axis_index("s") * 2 + lax.axis_index("c")
    pairs_per_w = n_pairs // N_WORKERS
    n_planes = 2 * pairs_per_w

    o_bufs = (out_a, out_b)
    o_sems = (s_oa, s_ob)

    zeros16 = jnp.zeros((L,), jnp.float32)

    def memset_plane(ref):
        def body(r, _):
            for u in range(ow // L):
                ref[r, pl.ds(u * L, L)] = zeros16
            return 0

        lax.fori_loop(0, oh, body, 0)

    def scatter_chunk(ref):
        def body(r, _):
            for u in range(w // L):
                idx = ind_c[r, pl.ds(u * L, L)]
                val = x_c[r, pl.ds(u * L, L)]
                q5 = jax.lax.shift_right_logical(idx, 5)
                row = jax.lax.shift_right_logical(q5 * 9363, 16)
                col = idx - row * ow
                plsc.store_scatter(ref, [row, col], val)
            return 0

        lax.fori_loop(0, CHUNK, body, 0)

    def plane_refs(q):
        k, t = q // 2, q % 2
        p = wid * pairs_per_w + k
        b = p // C_DIM
        c = p - b * C_DIM
        r = b * b_stride + t * t_stride + c
        return p, r

    def issue_chunk(q, c):
        p, r = plane_refs(q)
        hi = pltpu.async_copy(
            ind_hbm.at[p, pl.ds(c * CHUNK, CHUNK)], ind_c, s_i)
        hx = pltpu.async_copy(
            x_hbm.at[r, pl.ds(c * CHUNK, CHUNK)], x_c, s_x)
        return hi, hx

    pending = issue_chunk(0, 0)
    memset_plane(out_a)
    memset_plane(out_b)
    h_out = [None, None]

    for q in range(n_planes):
        obuf = o_bufs[q % 2]
        _, r = plane_refs(q)
        if h_out[q % 2] is not None:
            h_out[q % 2].wait()
            memset_plane(obuf)
        pending[0].wait()
        pending[1].wait()
        scatter_chunk(obuf)
        pending = issue_chunk(q, 1)
        pending[0].wait()
        pending[1].wait()
        scatter_chunk(obuf)
        if q + 1 < n_planes:
            pending = issue_chunk(q + 1, 0)
        h_out[q % 2] = pltpu.async_copy(obuf, out_hbm.at[r], o_sems[q % 2])

    h_out[0].wait()
    h_out[1].wait()


@jax.jit
def kernel(x, ind):
    bb, tt, cc, h, ww = x.shape
    x3 = x.reshape(bb * tt * cc, h, ww)
    ind3 = ind.reshape(bb * cc, h, ww)
    mesh = plsc.VectorSubcoreMesh(
        core_axis_name="c", subcore_axis_name="s", num_cores=2, num_subcores=16
    )
    run = pl.kernel(
        _unpool_body,
        out_type=jax.ShapeDtypeStruct((bb * tt * cc, 2 * h, 2 * ww), jnp.float32),
        mesh=mesh,
        scratch_types=[
            pltpu.VMEM((CHUNK, ww), jnp.int32),
            pltpu.VMEM((CHUNK, ww), jnp.float32),
            pltpu.VMEM((2 * h, 2 * ww), jnp.float32),
            pltpu.VMEM((2 * h, 2 * ww), jnp.float32),
            pltpu.SemaphoreType.DMA,
            pltpu.SemaphoreType.DMA,
            pltpu.SemaphoreType.DMA,
            pltpu.SemaphoreType.DMA,
        ],
        compiler_params=pltpu.CompilerParams(needs_layout_passes=False),
    )
    out = run(x3, ind3)
    return out.reshape(bb, tt, cc, 2 * h, 2 * ww)
